# baseline (speedup 1.0000x reference)
import functools
import numpy as np
import jax
import jax.numpy as jnp
from jax import lax
from jax.experimental import pallas as pl
from jax.experimental.pallas import tpu as pltpu
from jax.experimental.pallas import tpu_sc as plsc

H, DH, ROT = 16, 64, 32
NE, NEP = 7, 8
BASE = 10000.0


def _dot(a, b, dims):
    return lax.dot_general(a, b, (dims, ((), ())),
                           preferred_element_type=jnp.float32)


def _rot_perm():
    hrot = H * ROT
    i = np.arange(hrot)[:, None]
    j = np.arange(hrot)[None, :]
    same = (i // ROT) == (j // ROT)
    ci, cj = i % ROT, j % ROT
    p = np.where(same & (cj < ROT // 2) & (ci == cj + ROT // 2), -1.0, 0.0)
    p = p + np.where(same & (cj >= ROT // 2) & (ci == cj - ROT // 2), 1.0, 0.0)
    return jnp.asarray(p, jnp.float32)


def _proj_body(TB, T, x_ref, ln1_ref, wkv_ref, wq_ref, wku_ref, wqu_ref,
               wvu_ref, wrq_ref, wrk_ref, p_ref,
               qr_ref, qn_ref, kr_ref, kn_ref, v_ref):
    pid = pl.program_id(0)
    x = x_ref[...]
    h = x * lax.rsqrt(jnp.mean(x * x, axis=1, keepdims=True) + 1e-6) * ln1_ref[...]
    kvl = _dot(h, wkv_ref[...], ((1,), (1,)))
    ql = _dot(h, wq_ref[...], ((1,), (1,)))
    qn = _dot(ql, wqu_ref[...], ((1,), (1,)))
    kn = _dot(kvl, wku_ref[...], ((1,), (1,)))
    v = _dot(kvl, wvu_ref[...], ((1,), (1,)))
    qr = _dot(ql, wrq_ref[...], ((1,), (1,)))
    kr = _dot(h, wrk_ref[...], ((1,), (1,)))
    blocks_per_seq = T // TB
    t0 = (pid % blocks_per_seq) * TB
    trow = (t0 + lax.broadcasted_iota(jnp.int32, (TB, H * ROT), 0)).astype(jnp.float32)
    c = lax.broadcasted_iota(jnp.int32, (TB, H * ROT), 1)
    fidx = jnp.mod(c, ROT // 2).astype(jnp.float32)
    invf = jnp.exp(fidx * (-np.log(BASE) / (ROT // 2)))
    ang = trow * invf
    cos = jnp.cos(ang)
    sin = jnp.sin(ang)
    p = p_ref[...]
    qr = qr * cos + _dot(qr, p, ((1,), (0,))) * sin
    kr = kr * cos + _dot(kr, p, ((1,), (0,))) * sin
    for hh in range(H):
        qr_ref[0, hh] = qr[:, hh * ROT:(hh + 1) * ROT]
        kr_ref[0, hh] = kr[:, hh * ROT:(hh + 1) * ROT]
        qn_ref[0, hh] = qn[:, hh * DH + ROT:(hh + 1) * DH]
        kn_ref[0, hh] = kn[:, hh * DH + ROT:(hh + 1) * DH]
        v_ref[0, hh] = v[:, hh * DH:(hh + 1) * DH]


def _attn_body(BQ, T, qr_ref, qn_ref, kr_ref, kn_ref, v_ref, o_ref, s_scr):
    qi = pl.program_id(2)
    qr = qr_ref[0, 0]
    qn = qn_ref[0, 0]
    scale = 1.0 / np.sqrt(DH)

    def pass1(kb, m):
        kr = kr_ref[0, 0, pl.ds(kb * BQ, BQ), :]
        kn = kn_ref[0, 0, pl.ds(kb * BQ, BQ), :]
        s = (_dot(qr, kr, ((1,), (1,))) + _dot(qn, kn, ((1,), (1,)))) * scale
        row = qi * BQ + lax.broadcasted_iota(jnp.int32, (BQ, BQ), 0)
        col = kb * BQ + lax.broadcasted_iota(jnp.int32, (BQ, BQ), 1)
        s = jnp.where(col <= row, s, -jnp.inf)
        s_scr[:, pl.ds(kb * BQ, BQ)] = s
        return jnp.maximum(m, jnp.max(s, axis=1, keepdims=True))

    m = lax.fori_loop(0, qi + 1, pass1, jnp.full((BQ, 1), -jnp.inf, jnp.float32))

    def pass2(kb, carry):
        acc, den = carry
        p = jnp.exp(s_scr[:, pl.ds(kb * BQ, BQ)] - m)
        den = den + jnp.sum(p, axis=1, keepdims=True)
        acc = acc + _dot(p, v_ref[0, 0, pl.ds(kb * BQ, BQ), :], ((1,), (0,)))
        return acc, den

    acc, den = lax.fori_loop(
        0, qi + 1, pass2,
        (jnp.zeros((BQ, DH), jnp.float32), jnp.zeros((BQ, 1), jnp.float32)))
    o_ref[0, 0] = acc / den


def _post_body(x_ref, y_ref, wo_ref, ln2_ref, shg_ref, shu_ref, shd_ref,
               wr_ref, rb_ref, base_ref, h2_ref, lgt_ref):
    y = jnp.concatenate([y_ref[0, hh] for hh in range(H)], axis=1)
    x2 = x_ref[...] + _dot(y, wo_ref[...], ((1,), (1,)))
    h2 = x2 * lax.rsqrt(jnp.mean(x2 * x2, axis=1, keepdims=True) + 1e-6) * ln2_ref[...]
    h2_ref[...] = h2
    sg = _dot(h2, shg_ref[...], ((1,), (1,)))
    su = _dot(h2, shu_ref[...], ((1,), (1,)))
    act = sg * jax.nn.sigmoid(sg) * su
    base_ref[...] = x2 + _dot(act, shd_ref[...], ((1,), (1,)))
    lgt_ref[...] = _dot(wr_ref[...], h2, ((1,), (1,))) + rb_ref[...]


def _gemm_body(bm_ref, h2s_ref, rg_ref, ru_ref, rd_ref, out_ref):
    h2 = h2s_ref[...]
    g = _dot(h2, rg_ref[0], ((1,), (1,)))
    u = _dot(h2, ru_ref[0], ((1,), (1,)))
    act = g * jax.nn.sigmoid(g) * u
    out_ref[...] = _dot(act, rd_ref[0], ((1,), (1,)))


BLK = 256
NBLK = 39
CAP = NBLK * BLK
NW = 16
L = 16


def _lane_bcast(vec, j):
    idx = jnp.full((L,), j, jnp.int32)
    return vec.at[idx].get(mode="promise_in_bounds")


def _route_scatter(n, d):
    tpw = n // NW
    g_n = tpw // L
    mesh = plsc.VectorSubcoreMesh(core_axis_name="c", subcore_axis_name="s",
                                  num_cores=1, num_subcores=16)

    @functools.partial(
        pl.kernel,
        out_type=[
            jax.ShapeDtypeStruct((NW, g_n, L), jnp.int32),
            jax.ShapeDtypeStruct((NW, g_n, L), jnp.int32),
            jax.ShapeDtypeStruct((NW, g_n, L), jnp.float32),
            jax.ShapeDtypeStruct((NW, g_n, L), jnp.float32),
            jax.ShapeDtypeStruct((64,), jnp.int32),
            jax.ShapeDtypeStruct((CAP, d), jnp.float32),
            jax.ShapeDtypeStruct((NW, L), jnp.int32),
        ],
        mesh=mesh,
        scratch_types=[
            pltpu.VMEM((NE, n // NW), jnp.float32),
            pltpu.VMEM((g_n, L), jnp.int32),
            pltpu.VMEM((g_n, L), jnp.int32),
            pltpu.VMEM((g_n, L), jnp.float32),
            pltpu.VMEM((g_n, L), jnp.float32),
            pltpu.VMEM((g_n, L), jnp.int32),
            pltpu.VMEM((g_n, L), jnp.int32),
            pltpu.VMEM((L,), jnp.int32),
            pltpu.VMEM((64,), jnp.int32),
            pltpu.VMEM((L, d), jnp.float32),
            pltpu.VMEM((L,), jnp.int32),
            pltpu.VMEM((NW, L), jnp.int32),
            pltpu.SMEM((L,), jnp.int32),
            pltpu.SMEM((L,), jnp.int32),
            pltpu.SMEM((L,), jnp.int32),
        ],
    )
    def body(lgt_hbm, h2_hbm, pos1_hbm, pos2_hbm, w1_hbm, w2_hbm, bmap_hbm,
             h2s_hbm, cntx_hbm, lg_v, e1_v, e2_v, w1_v, w2_v, p1_v, p2_v,
             cnt_v, bmap_v, rows_v, idx_v, all_v, cnt_s, start_s, run_s):
        wid = lax.axis_index("s")
        lane = lax.iota(jnp.int32, L)

        pltpu.sync_copy(lgt_hbm.at[:, pl.ds(wid * tpw, tpw)], lg_v)

        def phase_a(g, c):
            p = [1.0 / (1.0 + jnp.exp(-lg_v[e, pl.ds(g * L, L)]))
                 for e in range(NE)]
            m1 = p[0]
            for e in range(1, NE):
                m1 = jnp.maximum(m1, p[e])
            i1f = jnp.full((L,), 99.0)
            for e in range(NE):
                i1f = jnp.minimum(i1f, jnp.where(p[e] == m1, float(e), 99.0))
            p2 = [jnp.where(i1f == float(e), -1.0, p[e]) for e in range(NE)]
            m2 = p2[0]
            for e in range(1, NE):
                m2 = jnp.maximum(m2, p2[e])
            i2f = jnp.full((L,), 99.0)
            for e in range(NE):
                i2f = jnp.minimum(i2f, jnp.where(p2[e] == m2, float(e), 99.0))
            ssum = m1 + m2
            e1_v[g] = i1f.astype(jnp.int32)
            e2_v[g] = i2f.astype(jnp.int32)
            w1_v[g] = m1 / ssum
            w2_v[g] = m2 / ssum
            return c

        lax.fori_loop(0, g_n, phase_a, 0)
        pltpu.sync_copy(w1_v, w1_hbm.at[wid])
        pltpu.sync_copy(w2_v, w2_hbm.at[wid])

        for e in range(L):
            cnt_s[e] = 0

        def hist(g, c):
            ev1 = e1_v[g]
            ev2 = e2_v[g]
            for j in range(L):
                ea = ev1[j]
                cnt_s[ea] = cnt_s[ea] + 1
                eb = ev2[j]
                cnt_s[eb] = cnt_s[eb] + 1
            return c

        lax.fori_loop(0, g_n, hist, 0)
        cv = jnp.zeros((L,), jnp.int32)
        for e in range(L):
            cv = jnp.where(lane == e, cnt_s[e], cv)
        cnt_v[...] = cv
        pltpu.sync_copy(cnt_v, cntx_hbm.at[wid])
        plsc.subcore_barrier()
        pltpu.sync_copy(cntx_hbm, all_v)

        all_rows = [all_v[s2] for s2 in range(NW)]
        seg_end_blk = []
        base_carry = jnp.int32(0)
        for e in range(NE):
            tot_e = jnp.int32(0)
            pre_e = jnp.int32(0)
            for s2 in range(NW):
                ce = all_rows[s2][e]
                tot_e = tot_e + ce
                pre_e = pre_e + jnp.where(s2 < wid, ce, 0)
            start_s[e] = base_carry + pre_e
            run_s[e] = 0
            padded_e = ((tot_e + (BLK - 1)) // BLK) * BLK
            base_carry = base_carry + padded_e
            seg_end_blk.append(base_carry // BLK)
        for bg in range(64 // L):
            bi = lane + bg * L
            acc = jnp.zeros((L,), jnp.int32)
            for e in range(NE):
                acc = acc + jnp.where(bi >= seg_end_blk[e], 1, 0)
            bmap_v[pl.ds(bg * L, L)] = jnp.minimum(acc, NE - 1)

        @pl.when(wid == 0)
        def _():
            pltpu.sync_copy(bmap_v, bmap_hbm)

        def posloop(g, c):
            ev1 = e1_v[g]
            ev2 = e2_v[g]
            pv1 = jnp.zeros((L,), jnp.int32)
            pv2 = jnp.zeros((L,), jnp.int32)
            for j in range(L):
                ea = ev1[j]
                pv1 = jnp.where(lane == j, start_s[ea] + run_s[ea], pv1)
                run_s[ea] = run_s[ea] + 1
                eb = ev2[j]
                pv2 = jnp.where(lane == j, start_s[eb] + run_s[eb], pv2)
                run_s[eb] = run_s[eb] + 1
            p1_v[g] = pv1
            p2_v[g] = pv2
            return c

        lax.fori_loop(0, g_n, posloop, 0)
        pltpu.sync_copy(p1_v, pos1_hbm.at[wid])
        pltpu.sync_copy(p2_v, pos2_hbm.at[wid])

        def scat(g, c):
            pltpu.sync_copy(h2_hbm.at[pl.ds(wid * tpw + g * L, L)], rows_v)
            idx_v[...] = jnp.clip(p1_v[g], 0, CAP - 1)
            pltpu.sync_copy(rows_v, h2s_hbm.at[idx_v])
            idx_v[...] = jnp.clip(p2_v[g], 0, CAP - 1)
            pltpu.sync_copy(rows_v, h2s_hbm.at[idx_v])
            return c

        lax.fori_loop(0, g_n, scat, 0)

    return body


def _combine(n, d):
    tpw = n // NW
    g_n = tpw // L
    mesh = plsc.VectorSubcoreMesh(core_axis_name="c", subcore_axis_name="s",
                                  num_cores=1, num_subcores=16)

    @functools.partial(
        pl.kernel,
        out_type=jax.ShapeDtypeStruct((n, d), jnp.float32),
        mesh=mesh,
        scratch_types=[
            pltpu.VMEM((g_n, L), jnp.int32),
            pltpu.VMEM((g_n, L), jnp.int32),
            pltpu.VMEM((g_n, L), jnp.float32),
            pltpu.VMEM((g_n, L), jnp.float32),
            pltpu.VMEM((L, d), jnp.float32),
            pltpu.VMEM((L, d), jnp.float32),
            pltpu.VMEM((L, d), jnp.float32),
            pltpu.VMEM((L,), jnp.int32),
        ],
    )
    def body(base_hbm, cout_hbm, pos1_hbm, pos2_hbm, w1_hbm, w2_hbm, out_hbm,
             p1_v, p2_v, w1_v, w2_v, r1_v, r2_v, acc_v, idx_v):
        wid = lax.axis_index("s")
        pltpu.sync_copy(pos1_hbm.at[wid], p1_v)
        pltpu.sync_copy(pos2_hbm.at[wid], p2_v)
        pltpu.sync_copy(w1_hbm.at[wid], w1_v)
        pltpu.sync_copy(w2_hbm.at[wid], w2_v)
        for g in range(g_n):
            idx_v[...] = jnp.clip(p1_v[g], 0, CAP - 1)
            pltpu.sync_copy(cout_hbm.at[idx_v], r1_v)
            idx_v[...] = jnp.clip(p2_v[g], 0, CAP - 1)
            pltpu.sync_copy(cout_hbm.at[idx_v], r2_v)
            pltpu.sync_copy(base_hbm.at[pl.ds(wid * tpw + g * L, L)], acc_v)

            wv1 = w1_v[g]
            wv2 = w2_v[g]
            for j in range(L):
                w1sc = wv1[j]
                w2sc = wv2[j]

                def col_body(cc, c2, j=j, w1sc=w1sc, w2sc=w2sc):
                    sl = pl.ds(cc * L, L)
                    acc_v[j, sl] = (acc_v[j, sl] + w1sc * r1_v[j, sl]
                                    + w2sc * r2_v[j, sl])
                    return c2

                lax.fori_loop(0, d // L, col_body, 0)
            pltpu.sync_copy(acc_v, out_hbm.at[pl.ds(wid * tpw + g * L, L)])

    return body


def kernel(x, ln1_w, ln2_w, w_kv_d, w_q_d, w_k_u, w_q_u, w_v_u, w_rope_q,
           w_rope_k, w_o, sh_gate, sh_up, sh_down, r_gate, r_up, r_down,
           w_router, routing_bias):
    b, t, d = x.shape
    n = b * t
    lat = w_kv_d.shape[0]
    i_dim = sh_gate.shape[0]
    xf = x.reshape(n, d)

    TB = min(256, t)
    bps = t // TB
    grid1 = (n // TB,)
    fullspec = lambda shape: pl.BlockSpec(shape, lambda i: (0,) * len(shape))
    rowspec = lambda w: pl.BlockSpec((TB, w), lambda i: (i, 0))
    hmspec = lambda w: pl.BlockSpec((1, H, TB, w),
                                    lambda i: (i // bps, 0, i % bps, 0))
    qr4, qn4, kr4, kn4, v4 = pl.pallas_call(
        functools.partial(_proj_body, TB, t),
        grid=grid1,
        in_specs=[
            rowspec(d), fullspec((1, d)), fullspec((lat, d)), fullspec((lat, d)),
            fullspec((H * DH, lat)), fullspec((H * DH, lat)), fullspec((H * DH, lat)),
            fullspec((H * ROT, lat)), fullspec((H * ROT, d)), fullspec((H * ROT, H * ROT)),
        ],
        out_specs=[hmspec(ROT), hmspec(DH - ROT), hmspec(ROT),
                   hmspec(DH - ROT), hmspec(DH)],
        out_shape=[
            jax.ShapeDtypeStruct((b, H, t, ROT), jnp.float32),
            jax.ShapeDtypeStruct((b, H, t, DH - ROT), jnp.float32),
            jax.ShapeDtypeStruct((b, H, t, ROT), jnp.float32),
            jax.ShapeDtypeStruct((b, H, t, DH - ROT), jnp.float32),
            jax.ShapeDtypeStruct((b, H, t, DH), jnp.float32),
        ],
    )(xf, ln1_w.reshape(1, d), w_kv_d, w_q_d, w_k_u, w_q_u, w_v_u,
      w_rope_q, w_rope_k, _rot_perm())

    BQ = min(256, t)
    y4 = pl.pallas_call(
        functools.partial(_attn_body, BQ, t),
        grid=(b, H, t // BQ),
        in_specs=[
            pl.BlockSpec((1, 1, BQ, ROT), lambda bb, hh, qi: (bb, hh, qi, 0)),
            pl.BlockSpec((1, 1, BQ, DH - ROT), lambda bb, hh, qi: (bb, hh, qi, 0)),
            pl.BlockSpec((1, 1, t, ROT), lambda bb, hh, qi: (bb, hh, 0, 0)),
            pl.BlockSpec((1, 1, t, DH - ROT), lambda bb, hh, qi: (bb, hh, 0, 0)),
            pl.BlockSpec((1, 1, t, DH), lambda bb, hh, qi: (bb, hh, 0, 0)),
        ],
        out_specs=pl.BlockSpec((1, 1, BQ, DH), lambda bb, hh, qi: (bb, hh, qi, 0)),
        out_shape=jax.ShapeDtypeStruct((b, H, t, DH), jnp.float32),
        scratch_shapes=[pltpu.VMEM((BQ, t), jnp.float32)],
        compiler_params=pltpu.CompilerParams(
            dimension_semantics=("parallel", "parallel", "parallel")),
    )(qr4, qn4, kr4, kn4, v4)

    TB3 = min(512, t)

    rowspec3 = lambda w: pl.BlockSpec((TB3, w), lambda i: (i, 0))
    base, h2, lgt = pl.pallas_call(
        _post_body,
        grid=(n // TB3,),
        in_specs=[
            rowspec3(d),
            pl.BlockSpec((1, H, TB3, DH),
                         lambda i: (i // (t // TB3), 0, i % (t // TB3), 0)),
            fullspec((d, d)), fullspec((1, d)),
            fullspec((i_dim, d)), fullspec((i_dim, d)), fullspec((d, i_dim)),
            fullspec((NE, d)), fullspec((NE, 1)),
        ],
        out_specs=[rowspec3(d), rowspec3(d),
                   pl.BlockSpec((NE, TB3), lambda i: (0, i))],
        out_shape=[
            jax.ShapeDtypeStruct((n, d), jnp.float32),
            jax.ShapeDtypeStruct((n, d), jnp.float32),
            jax.ShapeDtypeStruct((NE, n), jnp.float32),
        ],
    )(xf, y4, w_o, ln2_w.reshape(1, d), sh_gate, sh_up, sh_down,
      w_router, routing_bias.reshape(NE, 1))

    pos1, pos2, w1, w2, bmap, h2s, _cntx = _route_scatter(n, d)(lgt, h2)

    moe_gs = pltpu.PrefetchScalarGridSpec(
        num_scalar_prefetch=1,
        grid=(NBLK,),
        in_specs=[
            pl.BlockSpec((BLK, d), lambda i, bm: (i, 0)),
            pl.BlockSpec((1, i_dim, d),
                         lambda i, bm: (jnp.clip(bm[i], 0, NE - 1), 0, 0)),
            pl.BlockSpec((1, i_dim, d),
                         lambda i, bm: (jnp.clip(bm[i], 0, NE - 1), 0, 0)),
            pl.BlockSpec((1, d, i_dim),
                         lambda i, bm: (jnp.clip(bm[i], 0, NE - 1), 0, 0)),
        ],
        out_specs=pl.BlockSpec((BLK, d), lambda i, bm: (i, 0)),
    )
    cout = pl.pallas_call(
        _gemm_body,
        grid_spec=moe_gs,
        out_shape=jax.ShapeDtypeStruct((CAP, d), jnp.float32),
    )(bmap, h2s, r_gate, r_up, r_down)

    out = _combine(n, d)(base, cout, pos1, pos2, w1, w2)
    return out.reshape(b, t, d)


# speedup vs baseline: 1.1060x; 1.1060x over previous
import functools
import numpy as np
import jax
import jax.numpy as jnp
from jax import lax
from jax.experimental import pallas as pl
from jax.experimental.pallas import tpu as pltpu
from jax.experimental.pallas import tpu_sc as plsc

H, DH, ROT = 16, 64, 32
NE, NEP = 7, 8
BASE = 10000.0


def _dot(a, b, dims):
    return lax.dot_general(a, b, (dims, ((), ())),
                           preferred_element_type=jnp.float32)


def _dot16(a, b, dims):
    return lax.dot_general(a.astype(jnp.bfloat16), b.astype(jnp.bfloat16),
                           (dims, ((), ())),
                           preferred_element_type=jnp.float32)


def _rot_perm():
    hrot = H * ROT
    i = np.arange(hrot)[:, None]
    j = np.arange(hrot)[None, :]
    same = (i // ROT) == (j // ROT)
    ci, cj = i % ROT, j % ROT
    p = np.where(same & (cj < ROT // 2) & (ci == cj + ROT // 2), -1.0, 0.0)
    p = p + np.where(same & (cj >= ROT // 2) & (ci == cj - ROT // 2), 1.0, 0.0)
    return jnp.asarray(p, jnp.float32)


def _proj_body(TB, T, x_ref, ln1_ref, wkv_ref, wq_ref, wku_ref, wqu_ref,
               wvu_ref, wrq_ref, wrk_ref, p_ref,
               qr_ref, qn_ref, kr_ref, kn_ref, v_ref):
    pid = pl.program_id(0)
    x = x_ref[...]
    h = x * lax.rsqrt(jnp.mean(x * x, axis=1, keepdims=True) + 1e-6) * ln1_ref[...]
    kvl = _dot16(h, wkv_ref[...], ((1,), (1,)))
    ql = _dot16(h, wq_ref[...], ((1,), (1,)))
    qn = _dot16(ql, wqu_ref[...], ((1,), (1,)))
    kn = _dot16(kvl, wku_ref[...], ((1,), (1,)))
    v = _dot16(kvl, wvu_ref[...], ((1,), (1,)))
    qr = _dot16(ql, wrq_ref[...], ((1,), (1,)))
    kr = _dot16(h, wrk_ref[...], ((1,), (1,)))
    blocks_per_seq = T // TB
    t0 = (pid % blocks_per_seq) * TB
    trow = (t0 + lax.broadcasted_iota(jnp.int32, (TB, H * ROT), 0)).astype(jnp.float32)
    c = lax.broadcasted_iota(jnp.int32, (TB, H * ROT), 1)
    fidx = jnp.mod(c, ROT // 2).astype(jnp.float32)
    invf = jnp.exp(fidx * (-np.log(BASE) / (ROT // 2)))
    ang = trow * invf
    cos = jnp.cos(ang)
    sin = jnp.sin(ang)
    p = p_ref[...]
    qr = qr * cos + _dot(qr, p, ((1,), (0,))) * sin
    kr = kr * cos + _dot(kr, p, ((1,), (0,))) * sin
    for hh in range(H):
        qr_ref[0, hh] = qr[:, hh * ROT:(hh + 1) * ROT]
        kr_ref[0, hh] = kr[:, hh * ROT:(hh + 1) * ROT]
        qn_ref[0, hh] = qn[:, hh * DH + ROT:(hh + 1) * DH]
        kn_ref[0, hh] = kn[:, hh * DH + ROT:(hh + 1) * DH]
        v_ref[0, hh] = v[:, hh * DH:(hh + 1) * DH]


def _attn_body(BQ, T, qr_ref, qn_ref, kr_ref, kn_ref, v_ref, o_ref, s_scr):
    qi = pl.program_id(2)
    qr = qr_ref[0, 0]
    qn = qn_ref[0, 0]
    scale = 1.0 / np.sqrt(DH)

    def pass1(kb, m):
        kr = kr_ref[0, 0, pl.ds(kb * BQ, BQ), :]
        kn = kn_ref[0, 0, pl.ds(kb * BQ, BQ), :]
        s = (_dot16(qr, kr, ((1,), (1,))) + _dot16(qn, kn, ((1,), (1,)))) * scale
        row = qi * BQ + lax.broadcasted_iota(jnp.int32, (BQ, BQ), 0)
        col = kb * BQ + lax.broadcasted_iota(jnp.int32, (BQ, BQ), 1)
        s = jnp.where(col <= row, s, -jnp.inf)
        s_scr[:, pl.ds(kb * BQ, BQ)] = s
        return jnp.maximum(m, jnp.max(s, axis=1, keepdims=True))

    m = lax.fori_loop(0, qi + 1, pass1, jnp.full((BQ, 1), -jnp.inf, jnp.float32))

    def pass2(kb, carry):
        acc, den = carry
        p = jnp.exp(s_scr[:, pl.ds(kb * BQ, BQ)] - m)
        den = den + jnp.sum(p, axis=1, keepdims=True)
        acc = acc + _dot16(p, v_ref[0, 0, pl.ds(kb * BQ, BQ), :], ((1,), (0,)))
        return acc, den

    acc, den = lax.fori_loop(
        0, qi + 1, pass2,
        (jnp.zeros((BQ, DH), jnp.float32), jnp.zeros((BQ, 1), jnp.float32)))
    o_ref[0, 0] = acc / den


def _post_body(x_ref, y_ref, wo_ref, ln2_ref, shg_ref, shu_ref, shd_ref,
               wr_ref, rb_ref, base_ref, h2_ref, lgt_ref):
    y = jnp.concatenate([y_ref[0, hh] for hh in range(H)], axis=1)
    x2 = x_ref[...] + _dot16(y, wo_ref[...], ((1,), (1,)))
    h2 = x2 * lax.rsqrt(jnp.mean(x2 * x2, axis=1, keepdims=True) + 1e-6) * ln2_ref[...]
    h2_ref[...] = h2
    sg = _dot16(h2, shg_ref[...], ((1,), (1,)))
    su = _dot16(h2, shu_ref[...], ((1,), (1,)))
    act = sg * jax.nn.sigmoid(sg) * su
    base_ref[...] = x2 + _dot16(act, shd_ref[...], ((1,), (1,)))
    lgt_ref[...] = _dot(wr_ref[...], h2, ((1,), (1,))) + rb_ref[...]


def _gemm_body(bm_ref, h2s_ref, rg_ref, ru_ref, rd_ref, out_ref):
    h2 = h2s_ref[...]
    g = _dot16(h2, rg_ref[0], ((1,), (1,)))
    u = _dot16(h2, ru_ref[0], ((1,), (1,)))
    act = g * jax.nn.sigmoid(g) * u
    out_ref[...] = _dot16(act, rd_ref[0], ((1,), (1,)))


BLK = 256
NBLK = 39
CAP = NBLK * BLK
NW = 16
L = 16


def _lane_bcast(vec, j):
    idx = jnp.full((L,), j, jnp.int32)
    return vec.at[idx].get(mode="promise_in_bounds")


def _route_scatter(n, d):
    tpw = n // NW
    g_n = tpw // L
    mesh = plsc.VectorSubcoreMesh(core_axis_name="c", subcore_axis_name="s",
                                  num_cores=1, num_subcores=16)

    @functools.partial(
        pl.kernel,
        out_type=[
            jax.ShapeDtypeStruct((NW, g_n, L), jnp.int32),
            jax.ShapeDtypeStruct((NW, g_n, L), jnp.int32),
            jax.ShapeDtypeStruct((NW, g_n, L), jnp.float32),
            jax.ShapeDtypeStruct((NW, g_n, L), jnp.float32),
            jax.ShapeDtypeStruct((64,), jnp.int32),
            jax.ShapeDtypeStruct((CAP, d), jnp.float32),
            jax.ShapeDtypeStruct((NW, L), jnp.int32),
        ],
        mesh=mesh,
        scratch_types=[
            pltpu.VMEM((NE, n // NW), jnp.float32),
            pltpu.VMEM((g_n, L), jnp.int32),
            pltpu.VMEM((g_n, L), jnp.int32),
            pltpu.VMEM((g_n, L), jnp.float32),
            pltpu.VMEM((g_n, L), jnp.float32),
            pltpu.VMEM((g_n, L), jnp.int32),
            pltpu.VMEM((g_n, L), jnp.int32),
            pltpu.VMEM((L,), jnp.int32),
            pltpu.VMEM((64,), jnp.int32),
            pltpu.VMEM((L, d), jnp.float32),
            pltpu.VMEM((L,), jnp.int32),
            pltpu.VMEM((NW, L), jnp.int32),
            pltpu.SMEM((L,), jnp.int32),
            pltpu.SMEM((L,), jnp.int32),
            pltpu.SMEM((L,), jnp.int32),
        ],
    )
    def body(lgt_hbm, h2_hbm, pos1_hbm, pos2_hbm, w1_hbm, w2_hbm, bmap_hbm,
             h2s_hbm, cntx_hbm, lg_v, e1_v, e2_v, w1_v, w2_v, p1_v, p2_v,
             cnt_v, bmap_v, rows_v, idx_v, all_v, cnt_s, start_s, run_s):
        wid = lax.axis_index("s")
        lane = lax.iota(jnp.int32, L)

        pltpu.sync_copy(lgt_hbm.at[:, pl.ds(wid * tpw, tpw)], lg_v)

        def phase_a(g, c):
            p = [1.0 / (1.0 + jnp.exp(-lg_v[e, pl.ds(g * L, L)]))
                 for e in range(NE)]
            m1 = p[0]
            for e in range(1, NE):
                m1 = jnp.maximum(m1, p[e])
            i1f = jnp.full((L,), 99.0)
            for e in range(NE):
                i1f = jnp.minimum(i1f, jnp.where(p[e] == m1, float(e), 99.0))
            p2 = [jnp.where(i1f == float(e), -1.0, p[e]) for e in range(NE)]
            m2 = p2[0]
            for e in range(1, NE):
                m2 = jnp.maximum(m2, p2[e])
            i2f = jnp.full((L,), 99.0)
            for e in range(NE):
                i2f = jnp.minimum(i2f, jnp.where(p2[e] == m2, float(e), 99.0))
            ssum = m1 + m2
            e1_v[g] = i1f.astype(jnp.int32)
            e2_v[g] = i2f.astype(jnp.int32)
            w1_v[g] = m1 / ssum
            w2_v[g] = m2 / ssum
            return c

        lax.fori_loop(0, g_n, phase_a, 0)
        pltpu.sync_copy(w1_v, w1_hbm.at[wid])
        pltpu.sync_copy(w2_v, w2_hbm.at[wid])

        for e in range(L):
            cnt_s[e] = 0

        def hist(g, c):
            ev1 = e1_v[g]
            ev2 = e2_v[g]
            for j in range(L):
                ea = ev1[j]
                cnt_s[ea] = cnt_s[ea] + 1
                eb = ev2[j]
                cnt_s[eb] = cnt_s[eb] + 1
            return c

        lax.fori_loop(0, g_n, hist, 0)
        cv = jnp.zeros((L,), jnp.int32)
        for e in range(L):
            cv = jnp.where(lane == e, cnt_s[e], cv)
        cnt_v[...] = cv
        pltpu.sync_copy(cnt_v, cntx_hbm.at[wid])
        plsc.subcore_barrier()
        pltpu.sync_copy(cntx_hbm, all_v)

        all_rows = [all_v[s2] for s2 in range(NW)]
        seg_end_blk = []
        base_carry = jnp.int32(0)
        for e in range(NE):
            tot_e = jnp.int32(0)
            pre_e = jnp.int32(0)
            for s2 in range(NW):
                ce = all_rows[s2][e]
                tot_e = tot_e + ce
                pre_e = pre_e + jnp.where(s2 < wid, ce, 0)
            start_s[e] = base_carry + pre_e
            run_s[e] = 0
            padded_e = ((tot_e + (BLK - 1)) // BLK) * BLK
            base_carry = base_carry + padded_e
            seg_end_blk.append(base_carry // BLK)
        for bg in range(64 // L):
            bi = lane + bg * L
            acc = jnp.zeros((L,), jnp.int32)
            for e in range(NE):
                acc = acc + jnp.where(bi >= seg_end_blk[e], 1, 0)
            bmap_v[pl.ds(bg * L, L)] = jnp.minimum(acc, NE - 1)

        @pl.when(wid == 0)
        def _():
            pltpu.sync_copy(bmap_v, bmap_hbm)

        def posloop(g, c):
            ev1 = e1_v[g]
            ev2 = e2_v[g]
            pv1 = jnp.zeros((L,), jnp.int32)
            pv2 = jnp.zeros((L,), jnp.int32)
            for j in range(L):
                ea = ev1[j]
                pv1 = jnp.where(lane == j, start_s[ea] + run_s[ea], pv1)
                run_s[ea] = run_s[ea] + 1
                eb = ev2[j]
                pv2 = jnp.where(lane == j, start_s[eb] + run_s[eb], pv2)
                run_s[eb] = run_s[eb] + 1
            p1_v[g] = pv1
            p2_v[g] = pv2
            return c

        lax.fori_loop(0, g_n, posloop, 0)
        pltpu.sync_copy(p1_v, pos1_hbm.at[wid])
        pltpu.sync_copy(p2_v, pos2_hbm.at[wid])

        def scat(g, c):
            pltpu.sync_copy(h2_hbm.at[pl.ds(wid * tpw + g * L, L)], rows_v)
            idx_v[...] = jnp.clip(p1_v[g], 0, CAP - 1)
            pltpu.sync_copy(rows_v, h2s_hbm.at[idx_v])
            idx_v[...] = jnp.clip(p2_v[g], 0, CAP - 1)
            pltpu.sync_copy(rows_v, h2s_hbm.at[idx_v])
            return c

        lax.fori_loop(0, g_n, scat, 0)

    return body


def _combine(n, d):
    tpw = n // NW
    g_n = tpw // L
    mesh = plsc.VectorSubcoreMesh(core_axis_name="c", subcore_axis_name="s",
                                  num_cores=1, num_subcores=16)

    @functools.partial(
        pl.kernel,
        out_type=jax.ShapeDtypeStruct((n, d), jnp.float32),
        mesh=mesh,
        scratch_types=[
            pltpu.VMEM((g_n, L), jnp.int32),
            pltpu.VMEM((g_n, L), jnp.int32),
            pltpu.VMEM((g_n, L), jnp.float32),
            pltpu.VMEM((g_n, L), jnp.float32),
            pltpu.VMEM((L, d), jnp.float32),
            pltpu.VMEM((L, d), jnp.float32),
            pltpu.VMEM((L, d), jnp.float32),
            pltpu.VMEM((L,), jnp.int32),
        ],
    )
    def body(base_hbm, cout_hbm, pos1_hbm, pos2_hbm, w1_hbm, w2_hbm, out_hbm,
             p1_v, p2_v, w1_v, w2_v, r1_v, r2_v, acc_v, idx_v):
        wid = lax.axis_index("s")
        pltpu.sync_copy(pos1_hbm.at[wid], p1_v)
        pltpu.sync_copy(pos2_hbm.at[wid], p2_v)
        pltpu.sync_copy(w1_hbm.at[wid], w1_v)
        pltpu.sync_copy(w2_hbm.at[wid], w2_v)
        def group_body(g, c0):
            idx_v[...] = jnp.clip(p1_v[g], 0, CAP - 1)
            pltpu.sync_copy(cout_hbm.at[idx_v], r1_v)
            idx_v[...] = jnp.clip(p2_v[g], 0, CAP - 1)
            pltpu.sync_copy(cout_hbm.at[idx_v], r2_v)
            pltpu.sync_copy(base_hbm.at[pl.ds(wid * tpw + g * L, L)], acc_v)
            wv1 = w1_v[g]
            wv2 = w2_v[g]
            for j in range(L):
                w1sc = wv1[j]
                w2sc = wv2[j]

                def col_body(cb, c2, j=j, w1sc=w1sc, w2sc=w2sc):
                    for u in range(8):
                        sl = pl.ds(cb * (8 * L) + u * L, L)
                        acc_v[j, sl] = (acc_v[j, sl] + w1sc * r1_v[j, sl]
                                        + w2sc * r2_v[j, sl])
                    return c2

                lax.fori_loop(0, d // (8 * L), col_body, 0)
            pltpu.sync_copy(acc_v, out_hbm.at[pl.ds(wid * tpw + g * L, L)])
            return c0

        lax.fori_loop(0, g_n, group_body, 0)

    return body


def kernel(x, ln1_w, ln2_w, w_kv_d, w_q_d, w_k_u, w_q_u, w_v_u, w_rope_q,
           w_rope_k, w_o, sh_gate, sh_up, sh_down, r_gate, r_up, r_down,
           w_router, routing_bias):
    b, t, d = x.shape
    n = b * t
    lat = w_kv_d.shape[0]
    i_dim = sh_gate.shape[0]
    xf = x.reshape(n, d)

    TB = min(256, t)
    bps = t // TB
    grid1 = (n // TB,)
    fullspec = lambda shape: pl.BlockSpec(shape, lambda i: (0,) * len(shape))
    rowspec = lambda w: pl.BlockSpec((TB, w), lambda i: (i, 0))
    hmspec = lambda w: pl.BlockSpec((1, H, TB, w),
                                    lambda i: (i // bps, 0, i % bps, 0))
    qr4, qn4, kr4, kn4, v4 = pl.pallas_call(
        functools.partial(_proj_body, TB, t),
        grid=grid1,
        in_specs=[
            rowspec(d), fullspec((1, d)), fullspec((lat, d)), fullspec((lat, d)),
            fullspec((H * DH, lat)), fullspec((H * DH, lat)), fullspec((H * DH, lat)),
            fullspec((H * ROT, lat)), fullspec((H * ROT, d)), fullspec((H * ROT, H * ROT)),
        ],
        out_specs=[hmspec(ROT), hmspec(DH - ROT), hmspec(ROT),
                   hmspec(DH - ROT), hmspec(DH)],
        out_shape=[
            jax.ShapeDtypeStruct((b, H, t, ROT), jnp.float32),
            jax.ShapeDtypeStruct((b, H, t, DH - ROT), jnp.float32),
            jax.ShapeDtypeStruct((b, H, t, ROT), jnp.float32),
            jax.ShapeDtypeStruct((b, H, t, DH - ROT), jnp.float32),
            jax.ShapeDtypeStruct((b, H, t, DH), jnp.float32),
        ],
    )(xf, ln1_w.reshape(1, d), w_kv_d, w_q_d, w_k_u, w_q_u, w_v_u,
      w_rope_q, w_rope_k, _rot_perm())

    BQ = min(256, t)
    y4 = pl.pallas_call(
        functools.partial(_attn_body, BQ, t),
        grid=(b, H, t // BQ),
        in_specs=[
            pl.BlockSpec((1, 1, BQ, ROT), lambda bb, hh, qi: (bb, hh, qi, 0)),
            pl.BlockSpec((1, 1, BQ, DH - ROT), lambda bb, hh, qi: (bb, hh, qi, 0)),
            pl.BlockSpec((1, 1, t, ROT), lambda bb, hh, qi: (bb, hh, 0, 0)),
            pl.BlockSpec((1, 1, t, DH - ROT), lambda bb, hh, qi: (bb, hh, 0, 0)),
            pl.BlockSpec((1, 1, t, DH), lambda bb, hh, qi: (bb, hh, 0, 0)),
        ],
        out_specs=pl.BlockSpec((1, 1, BQ, DH), lambda bb, hh, qi: (bb, hh, qi, 0)),
        out_shape=jax.ShapeDtypeStruct((b, H, t, DH), jnp.float32),
        scratch_shapes=[pltpu.VMEM((BQ, t), jnp.float32)],
        compiler_params=pltpu.CompilerParams(
            dimension_semantics=("parallel", "parallel", "parallel")),
    )(qr4, qn4, kr4, kn4, v4)

    TB3 = min(512, t)

    rowspec3 = lambda w: pl.BlockSpec((TB3, w), lambda i: (i, 0))
    base, h2, lgt = pl.pallas_call(
        _post_body,
        grid=(n // TB3,),
        in_specs=[
            rowspec3(d),
            pl.BlockSpec((1, H, TB3, DH),
                         lambda i: (i // (t // TB3), 0, i % (t // TB3), 0)),
            fullspec((d, d)), fullspec((1, d)),
            fullspec((i_dim, d)), fullspec((i_dim, d)), fullspec((d, i_dim)),
            fullspec((NE, d)), fullspec((NE, 1)),
        ],
        out_specs=[rowspec3(d), rowspec3(d),
                   pl.BlockSpec((NE, TB3), lambda i: (0, i))],
        out_shape=[
            jax.ShapeDtypeStruct((n, d), jnp.float32),
            jax.ShapeDtypeStruct((n, d), jnp.float32),
            jax.ShapeDtypeStruct((NE, n), jnp.float32),
        ],
    )(xf, y4, w_o, ln2_w.reshape(1, d), sh_gate, sh_up, sh_down,
      w_router, routing_bias.reshape(NE, 1))

    pos1, pos2, w1, w2, bmap, h2s, _cntx = _route_scatter(n, d)(lgt, h2)

    moe_gs = pltpu.PrefetchScalarGridSpec(
        num_scalar_prefetch=1,
        grid=(NBLK,),
        in_specs=[
            pl.BlockSpec((BLK, d), lambda i, bm: (i, 0)),
            pl.BlockSpec((1, i_dim, d),
                         lambda i, bm: (jnp.clip(bm[i], 0, NE - 1), 0, 0)),
            pl.BlockSpec((1, i_dim, d),
                         lambda i, bm: (jnp.clip(bm[i], 0, NE - 1), 0, 0)),
            pl.BlockSpec((1, d, i_dim),
                         lambda i, bm: (jnp.clip(bm[i], 0, NE - 1), 0, 0)),
        ],
        out_specs=pl.BlockSpec((BLK, d), lambda i, bm: (i, 0)),
    )
    cout = pl.pallas_call(
        _gemm_body,
        grid_spec=moe_gs,
        out_shape=jax.ShapeDtypeStruct((CAP, d), jnp.float32),
    )(bmap, h2s, r_gate, r_up, r_down)

    out = _combine(n, d)(base, cout, pos1, pos2, w1, w2)
    return out.reshape(b, t, d)


# speedup vs baseline: 1.7080x; 1.5443x over previous
import functools
import numpy as np
import jax
import jax.numpy as jnp
from jax import lax
from jax.experimental import pallas as pl
from jax.experimental.pallas import tpu as pltpu
from jax.experimental.pallas import tpu_sc as plsc

H, DH, ROT = 16, 64, 32
NE, NEP = 7, 8
BASE = 10000.0


def _dot(a, b, dims):
    return lax.dot_general(a, b, (dims, ((), ())),
                           preferred_element_type=jnp.float32)


def _dot16(a, b, dims):
    return lax.dot_general(a.astype(jnp.bfloat16), b.astype(jnp.bfloat16),
                           (dims, ((), ())),
                           preferred_element_type=jnp.float32)


def _rot_perm():
    hrot = H * ROT
    i = np.arange(hrot)[:, None]
    j = np.arange(hrot)[None, :]
    same = (i // ROT) == (j // ROT)
    ci, cj = i % ROT, j % ROT
    p = np.where(same & (cj < ROT // 2) & (ci == cj + ROT // 2), -1.0, 0.0)
    p = p + np.where(same & (cj >= ROT // 2) & (ci == cj - ROT // 2), 1.0, 0.0)
    return jnp.asarray(p, jnp.float32)


def _proj_body(TB, T, x_ref, ln1_ref, wkv_ref, wq_ref, wku_ref, wqu_ref,
               wvu_ref, wrq_ref, wrk_ref, p_ref,
               qr_ref, qn_ref, kr_ref, kn_ref, v_ref):
    pid = pl.program_id(0)
    x = x_ref[...]
    h = x * lax.rsqrt(jnp.mean(x * x, axis=1, keepdims=True) + 1e-6) * ln1_ref[...]
    kvl = _dot16(h, wkv_ref[...], ((1,), (1,)))
    ql = _dot16(h, wq_ref[...], ((1,), (1,)))
    qn = _dot16(ql, wqu_ref[...], ((1,), (1,)))
    kn = _dot16(kvl, wku_ref[...], ((1,), (1,)))
    v = _dot16(kvl, wvu_ref[...], ((1,), (1,)))
    qr = _dot16(ql, wrq_ref[...], ((1,), (1,)))
    kr = _dot16(h, wrk_ref[...], ((1,), (1,)))
    blocks_per_seq = T // TB
    t0 = (pid % blocks_per_seq) * TB
    trow = (t0 + lax.broadcasted_iota(jnp.int32, (TB, H * ROT), 0)).astype(jnp.float32)
    c = lax.broadcasted_iota(jnp.int32, (TB, H * ROT), 1)
    fidx = jnp.mod(c, ROT // 2).astype(jnp.float32)
    invf = jnp.exp(fidx * (-np.log(BASE) / (ROT // 2)))
    ang = trow * invf
    cos = jnp.cos(ang)
    sin = jnp.sin(ang)
    p = p_ref[...]
    qr = qr * cos + _dot(qr, p, ((1,), (0,))) * sin
    kr = kr * cos + _dot(kr, p, ((1,), (0,))) * sin
    for hh in range(H):
        qr_ref[0, hh] = qr[:, hh * ROT:(hh + 1) * ROT]
        kr_ref[0, hh] = kr[:, hh * ROT:(hh + 1) * ROT]
        qn_ref[0, hh] = qn[:, hh * DH + ROT:(hh + 1) * DH]
        kn_ref[0, hh] = kn[:, hh * DH + ROT:(hh + 1) * DH]
        v_ref[0, hh] = v[:, hh * DH:(hh + 1) * DH]


def _attn_body(BQ, T, qr_ref, qn_ref, kr_ref, kn_ref, v_ref, o_ref):
    qi = pl.program_id(2)
    qr = qr_ref[0, 0]
    qn = qn_ref[0, 0]
    scale = 1.0 / np.sqrt(DH)

    def chunk(kb, carry, masked):
        m, acc, den = carry
        kr = kr_ref[0, 0, pl.ds(kb * BQ, BQ), :]
        kn = kn_ref[0, 0, pl.ds(kb * BQ, BQ), :]
        s = (_dot16(qr, kr, ((1,), (1,))) + _dot16(qn, kn, ((1,), (1,)))) * scale
        if masked:
            row = lax.broadcasted_iota(jnp.int32, (BQ, BQ), 0)
            col = lax.broadcasted_iota(jnp.int32, (BQ, BQ), 1)
            s = jnp.where(col <= row, s, -jnp.inf)
        m2 = jnp.maximum(m, jnp.max(s, axis=1, keepdims=True))
        r = jnp.exp(m - m2)
        p = jnp.exp(s - m2)
        den = den * r + jnp.sum(p, axis=1, keepdims=True)
        acc = acc * r + _dot16(p, v_ref[0, 0, pl.ds(kb * BQ, BQ), :],
                               ((1,), (0,)))
        return m2, acc, den

    carry0 = (jnp.full((BQ, 1), -jnp.inf, jnp.float32),
              jnp.zeros((BQ, DH), jnp.float32),
              jnp.zeros((BQ, 1), jnp.float32))
    carry = lax.fori_loop(0, qi, lambda kb, c: chunk(kb, c, False), carry0)
    _, acc, den = chunk(qi, carry, True)
    o_ref[0, 0] = acc / den


def _post_body(x_ref, y_ref, wo_ref, ln2_ref, shg_ref, shu_ref, shd_ref,
               wr_ref, rb_ref, base_ref, h2_ref, lgt_ref):
    y = jnp.concatenate([y_ref[0, hh] for hh in range(H)], axis=1)
    x2 = x_ref[...] + _dot16(y, wo_ref[...], ((1,), (1,)))
    h2 = x2 * lax.rsqrt(jnp.mean(x2 * x2, axis=1, keepdims=True) + 1e-6) * ln2_ref[...]
    h2_ref[...] = h2
    sg = _dot16(h2, shg_ref[...], ((1,), (1,)))
    su = _dot16(h2, shu_ref[...], ((1,), (1,)))
    act = sg * jax.nn.sigmoid(sg) * su
    base_ref[...] = x2 + _dot16(act, shd_ref[...], ((1,), (1,)))
    lgt_ref[...] = _dot(wr_ref[...], h2, ((1,), (1,))) + rb_ref[...]


def _gemm_body(bm_ref, h2s_ref, rg_ref, ru_ref, rd_ref, out_ref):
    h2 = h2s_ref[...]
    g = _dot16(h2, rg_ref[0], ((1,), (1,)))
    u = _dot16(h2, ru_ref[0], ((1,), (1,)))
    act = g * jax.nn.sigmoid(g) * u
    out_ref[...] = _dot16(act, rd_ref[0], ((1,), (1,)))


BLK = 256
NBLK = 39
CAP = NBLK * BLK
NW = 16
L = 16


def _lane_bcast(vec, j):
    idx = jnp.full((L,), j, jnp.int32)
    return vec.at[idx].get(mode="promise_in_bounds")


def _route_scatter(n, d):
    tpw = n // NW
    g_n = tpw // L
    mesh = plsc.VectorSubcoreMesh(core_axis_name="c", subcore_axis_name="s",
                                  num_cores=1, num_subcores=16)

    @functools.partial(
        pl.kernel,
        out_type=[
            jax.ShapeDtypeStruct((NW, g_n, L), jnp.int32),
            jax.ShapeDtypeStruct((NW, g_n, L), jnp.int32),
            jax.ShapeDtypeStruct((NW, g_n, L), jnp.float32),
            jax.ShapeDtypeStruct((NW, g_n, L), jnp.float32),
            jax.ShapeDtypeStruct((64,), jnp.int32),
            jax.ShapeDtypeStruct((CAP, d), jnp.float32),
            jax.ShapeDtypeStruct((NW, L), jnp.int32),
        ],
        mesh=mesh,
        scratch_types=[
            pltpu.VMEM((NE, n // NW), jnp.float32),
            pltpu.VMEM((g_n, L), jnp.int32),
            pltpu.VMEM((g_n, L), jnp.int32),
            pltpu.VMEM((g_n, L), jnp.float32),
            pltpu.VMEM((g_n, L), jnp.float32),
            pltpu.VMEM((g_n, L), jnp.int32),
            pltpu.VMEM((g_n, L), jnp.int32),
            pltpu.VMEM((L,), jnp.int32),
            pltpu.VMEM((64,), jnp.int32),
            pltpu.VMEM((L, d), jnp.float32),
            pltpu.VMEM((L,), jnp.int32),
            pltpu.VMEM((NW, L), jnp.int32),
            pltpu.SMEM((L,), jnp.int32),
            pltpu.SMEM((L,), jnp.int32),
            pltpu.SMEM((L,), jnp.int32),
        ],
    )
    def body(lgt_hbm, h2_hbm, pos1_hbm, pos2_hbm, w1_hbm, w2_hbm, bmap_hbm,
             h2s_hbm, cntx_hbm, lg_v, e1_v, e2_v, w1_v, w2_v, p1_v, p2_v,
             cnt_v, bmap_v, rows_v, idx_v, all_v, cnt_s, start_s, run_s):
        wid = lax.axis_index("s")
        lane = lax.iota(jnp.int32, L)

        pltpu.sync_copy(lgt_hbm.at[:, pl.ds(wid * tpw, tpw)], lg_v)

        def phase_a(g, c):
            p = [1.0 / (1.0 + jnp.exp(-lg_v[e, pl.ds(g * L, L)]))
                 for e in range(NE)]
            m1 = p[0]
            for e in range(1, NE):
                m1 = jnp.maximum(m1, p[e])
            i1f = jnp.full((L,), 99.0)
            for e in range(NE):
                i1f = jnp.minimum(i1f, jnp.where(p[e] == m1, float(e), 99.0))
            p2 = [jnp.where(i1f == float(e), -1.0, p[e]) for e in range(NE)]
            m2 = p2[0]
            for e in range(1, NE):
                m2 = jnp.maximum(m2, p2[e])
            i2f = jnp.full((L,), 99.0)
            for e in range(NE):
                i2f = jnp.minimum(i2f, jnp.where(p2[e] == m2, float(e), 99.0))
            ssum = m1 + m2
            e1_v[g] = i1f.astype(jnp.int32)
            e2_v[g] = i2f.astype(jnp.int32)
            w1_v[g] = m1 / ssum
            w2_v[g] = m2 / ssum
            return c

        lax.fori_loop(0, g_n, phase_a, 0)
        pltpu.sync_copy(w1_v, w1_hbm.at[wid])
        pltpu.sync_copy(w2_v, w2_hbm.at[wid])

        for e in range(L):
            cnt_s[e] = 0

        def hist(g, c):
            ev1 = e1_v[g]
            ev2 = e2_v[g]
            for j in range(L):
                ea = ev1[j]
                cnt_s[ea] = cnt_s[ea] + 1
                eb = ev2[j]
                cnt_s[eb] = cnt_s[eb] + 1
            return c

        lax.fori_loop(0, g_n, hist, 0)
        cv = jnp.zeros((L,), jnp.int32)
        for e in range(L):
            cv = jnp.where(lane == e, cnt_s[e], cv)
        cnt_v[...] = cv
        pltpu.sync_copy(cnt_v, cntx_hbm.at[wid])
        plsc.subcore_barrier()
        pltpu.sync_copy(cntx_hbm, all_v)

        all_rows = [all_v[s2] for s2 in range(NW)]
        seg_end_blk = []
        base_carry = jnp.int32(0)
        for e in range(NE):
            tot_e = jnp.int32(0)
            pre_e = jnp.int32(0)
            for s2 in range(NW):
                ce = all_rows[s2][e]
                tot_e = tot_e + ce
                pre_e = pre_e + jnp.where(s2 < wid, ce, 0)
            start_s[e] = base_carry + pre_e
            run_s[e] = 0
            padded_e = ((tot_e + (BLK - 1)) // BLK) * BLK
            base_carry = base_carry + padded_e
            seg_end_blk.append(base_carry // BLK)
        for bg in range(64 // L):
            bi = lane + bg * L
            acc = jnp.zeros((L,), jnp.int32)
            for e in range(NE):
                acc = acc + jnp.where(bi >= seg_end_blk[e], 1, 0)
            bmap_v[pl.ds(bg * L, L)] = jnp.minimum(acc, NE - 1)

        @pl.when(wid == 0)
        def _():
            pltpu.sync_copy(bmap_v, bmap_hbm)

        def posloop(g, c):
            ev1 = e1_v[g]
            ev2 = e2_v[g]
            pv1 = jnp.zeros((L,), jnp.int32)
            pv2 = jnp.zeros((L,), jnp.int32)
            for j in range(L):
                ea = ev1[j]
                pv1 = jnp.where(lane == j, start_s[ea] + run_s[ea], pv1)
                run_s[ea] = run_s[ea] + 1
                eb = ev2[j]
                pv2 = jnp.where(lane == j, start_s[eb] + run_s[eb], pv2)
                run_s[eb] = run_s[eb] + 1
            p1_v[g] = pv1
            p2_v[g] = pv2
            return c

        lax.fori_loop(0, g_n, posloop, 0)
        pltpu.sync_copy(p1_v, pos1_hbm.at[wid])
        pltpu.sync_copy(p2_v, pos2_hbm.at[wid])

        def scat(g, c):
            pltpu.sync_copy(h2_hbm.at[pl.ds(wid * tpw + g * L, L)], rows_v)
            idx_v[...] = jnp.clip(p1_v[g], 0, CAP - 1)
            pltpu.sync_copy(rows_v, h2s_hbm.at[idx_v])
            idx_v[...] = jnp.clip(p2_v[g], 0, CAP - 1)
            pltpu.sync_copy(rows_v, h2s_hbm.at[idx_v])
            return c

        lax.fori_loop(0, g_n, scat, 0)

    return body


def _combine(n, d):
    tpw = n // NW
    g_n = tpw // L
    mesh = plsc.VectorSubcoreMesh(core_axis_name="c", subcore_axis_name="s",
                                  num_cores=1, num_subcores=16)

    @functools.partial(
        pl.kernel,
        out_type=jax.ShapeDtypeStruct((n, d), jnp.float32),
        mesh=mesh,
        scratch_types=[
            pltpu.VMEM((g_n, L), jnp.int32),
            pltpu.VMEM((g_n, L), jnp.int32),
            pltpu.VMEM((g_n, L), jnp.float32),
            pltpu.VMEM((g_n, L), jnp.float32),
            pltpu.VMEM((L, d), jnp.float32),
            pltpu.VMEM((L, d), jnp.float32),
            pltpu.VMEM((L, d), jnp.float32),
            pltpu.VMEM((L,), jnp.int32),
        ],
    )
    def body(base_hbm, cout_hbm, pos1_hbm, pos2_hbm, w1_hbm, w2_hbm, out_hbm,
             p1_v, p2_v, w1_v, w2_v, r1_v, r2_v, acc_v, idx_v):
        wid = lax.axis_index("s")
        pltpu.sync_copy(pos1_hbm.at[wid], p1_v)
        pltpu.sync_copy(pos2_hbm.at[wid], p2_v)
        pltpu.sync_copy(w1_hbm.at[wid], w1_v)
        pltpu.sync_copy(w2_hbm.at[wid], w2_v)
        def group_body(g, c0):
            idx_v[...] = jnp.clip(p1_v[g], 0, CAP - 1)
            pltpu.sync_copy(cout_hbm.at[idx_v], r1_v)
            idx_v[...] = jnp.clip(p2_v[g], 0, CAP - 1)
            pltpu.sync_copy(cout_hbm.at[idx_v], r2_v)
            pltpu.sync_copy(base_hbm.at[pl.ds(wid * tpw + g * L, L)], acc_v)
            wv1 = w1_v[g]
            wv2 = w2_v[g]
            for j in range(L):
                w1sc = wv1[j]
                w2sc = wv2[j]

                def col_body(cb, c2, j=j, w1sc=w1sc, w2sc=w2sc):
                    for u in range(8):
                        sl = pl.ds(cb * (8 * L) + u * L, L)
                        acc_v[j, sl] = (acc_v[j, sl] + w1sc * r1_v[j, sl]
                                        + w2sc * r2_v[j, sl])
                    return c2

                lax.fori_loop(0, d // (8 * L), col_body, 0)
            pltpu.sync_copy(acc_v, out_hbm.at[pl.ds(wid * tpw + g * L, L)])
            return c0

        lax.fori_loop(0, g_n, group_body, 0)

    return body


def kernel(x, ln1_w, ln2_w, w_kv_d, w_q_d, w_k_u, w_q_u, w_v_u, w_rope_q,
           w_rope_k, w_o, sh_gate, sh_up, sh_down, r_gate, r_up, r_down,
           w_router, routing_bias):
    b, t, d = x.shape
    n = b * t
    lat = w_kv_d.shape[0]
    i_dim = sh_gate.shape[0]
    xf = x.reshape(n, d)

    TB = min(256, t)
    bps = t // TB
    grid1 = (n // TB,)
    fullspec = lambda shape: pl.BlockSpec(shape, lambda i: (0,) * len(shape))
    rowspec = lambda w: pl.BlockSpec((TB, w), lambda i: (i, 0))
    hmspec = lambda w: pl.BlockSpec((1, H, TB, w),
                                    lambda i: (i // bps, 0, i % bps, 0))
    qr4, qn4, kr4, kn4, v4 = pl.pallas_call(
        functools.partial(_proj_body, TB, t),
        grid=grid1,
        in_specs=[
            rowspec(d), fullspec((1, d)), fullspec((lat, d)), fullspec((lat, d)),
            fullspec((H * DH, lat)), fullspec((H * DH, lat)), fullspec((H * DH, lat)),
            fullspec((H * ROT, lat)), fullspec((H * ROT, d)), fullspec((H * ROT, H * ROT)),
        ],
        out_specs=[hmspec(ROT), hmspec(DH - ROT), hmspec(ROT),
                   hmspec(DH - ROT), hmspec(DH)],
        out_shape=[
            jax.ShapeDtypeStruct((b, H, t, ROT), jnp.float32),
            jax.ShapeDtypeStruct((b, H, t, DH - ROT), jnp.float32),
            jax.ShapeDtypeStruct((b, H, t, ROT), jnp.float32),
            jax.ShapeDtypeStruct((b, H, t, DH - ROT), jnp.float32),
            jax.ShapeDtypeStruct((b, H, t, DH), jnp.float32),
        ],
    )(xf, ln1_w.reshape(1, d), w_kv_d, w_q_d, w_k_u, w_q_u, w_v_u,
      w_rope_q, w_rope_k, _rot_perm())

    BQ = min(512, t)
    y4 = pl.pallas_call(
        functools.partial(_attn_body, BQ, t),
        grid=(b, H, t // BQ),
        in_specs=[
            pl.BlockSpec((1, 1, BQ, ROT), lambda bb, hh, qi: (bb, hh, qi, 0)),
            pl.BlockSpec((1, 1, BQ, DH - ROT), lambda bb, hh, qi: (bb, hh, qi, 0)),
            pl.BlockSpec((1, 1, t, ROT), lambda bb, hh, qi: (bb, hh, 0, 0)),
            pl.BlockSpec((1, 1, t, DH - ROT), lambda bb, hh, qi: (bb, hh, 0, 0)),
            pl.BlockSpec((1, 1, t, DH), lambda bb, hh, qi: (bb, hh, 0, 0)),
        ],
        out_specs=pl.BlockSpec((1, 1, BQ, DH), lambda bb, hh, qi: (bb, hh, qi, 0)),
        out_shape=jax.ShapeDtypeStruct((b, H, t, DH), jnp.float32),
        compiler_params=pltpu.CompilerParams(
            dimension_semantics=("parallel", "parallel", "parallel")),
    )(qr4, qn4, kr4, kn4, v4)

    TB3 = min(512, t)

    rowspec3 = lambda w: pl.BlockSpec((TB3, w), lambda i: (i, 0))
    base, h2, lgt = pl.pallas_call(
        _post_body,
        grid=(n // TB3,),
        in_specs=[
            rowspec3(d),
            pl.BlockSpec((1, H, TB3, DH),
                         lambda i: (i // (t // TB3), 0, i % (t // TB3), 0)),
            fullspec((d, d)), fullspec((1, d)),
            fullspec((i_dim, d)), fullspec((i_dim, d)), fullspec((d, i_dim)),
            fullspec((NE, d)), fullspec((NE, 1)),
        ],
        out_specs=[rowspec3(d), rowspec3(d),
                   pl.BlockSpec((NE, TB3), lambda i: (0, i))],
        out_shape=[
            jax.ShapeDtypeStruct((n, d), jnp.float32),
            jax.ShapeDtypeStruct((n, d), jnp.float32),
            jax.ShapeDtypeStruct((NE, n), jnp.float32),
        ],
    )(xf, y4, w_o, ln2_w.reshape(1, d), sh_gate, sh_up, sh_down,
      w_router, routing_bias.reshape(NE, 1))

    pos1, pos2, w1, w2, bmap, h2s, _cntx = _route_scatter(n, d)(lgt, h2)

    moe_gs = pltpu.PrefetchScalarGridSpec(
        num_scalar_prefetch=1,
        grid=(NBLK,),
        in_specs=[
            pl.BlockSpec((BLK, d), lambda i, bm: (i, 0)),
            pl.BlockSpec((1, i_dim, d),
                         lambda i, bm: (jnp.clip(bm[i], 0, NE - 1), 0, 0)),
            pl.BlockSpec((1, i_dim, d),
                         lambda i, bm: (jnp.clip(bm[i], 0, NE - 1), 0, 0)),
            pl.BlockSpec((1, d, i_dim),
                         lambda i, bm: (jnp.clip(bm[i], 0, NE - 1), 0, 0)),
        ],
        out_specs=pl.BlockSpec((BLK, d), lambda i, bm: (i, 0)),
    )
    cout = pl.pallas_call(
        _gemm_body,
        grid_spec=moe_gs,
        out_shape=jax.ShapeDtypeStruct((CAP, d), jnp.float32),
    )(bmap, h2s, r_gate, r_up, r_down)

    out = _combine(n, d)(base, cout, pos1, pos2, w1, w2)
    return out.reshape(b, t, d)
